# baseline (speedup 1.0000x reference)
import functools
import math

import jax
import jax.numpy as jnp
from jax.experimental import pallas as pl
from jax.experimental.pallas import tpu as pltpu

_SCALE = 3.0
_SHIFTS = 16
_BLOCK = 2048
_C = 2.0 * _SCALE * _SCALE * math.log2(math.e)
_LN2 = math.log(2.0)


def _unit_rows(x):
    n = jnp.sqrt(jnp.sum(x * x, axis=-1, keepdims=True))
    return x / jnp.maximum(n, 1e-12)


def _nca_body(xa_ref, xb_ref, p_ref, ta_ref, tb_ref, t2b_ref, out_ref,
              pn_ref, ones_ref, g_ref, virt_ref, acc_ref,
              *, nblk, block, ncls):
    s = pl.program_id(0)

    @pl.when(s == 0)
    def _init():
        acc_ref[0] = 0.0
        acc_ref[1] = 0.0
        pn_ref[:, :] = _unit_rows(p_ref[:, :]).astype(jnp.bfloat16)
        ones_ref[:, :] = jnp.ones_like(ones_ref)

    cols = jax.lax.broadcasted_iota(jnp.int32, (block, ncls), 1)

    def _rowsum(a):
        return jax.lax.dot_general(
            a.astype(jnp.bfloat16), ones_ref[:, :], (((1,), (0,)), ((), ())),
            preferred_element_type=jnp.float32)[:, :1]

    def _y_lse(x, sqsum):
        inv = _C / jnp.maximum(jnp.sqrt(sqsum), 1e-12)
        y = jax.lax.dot_general(
            (x * inv).astype(jnp.bfloat16), pn_ref[:, :],
            (((1,), (1,)), ((), ())), preferred_element_type=jnp.float32)
        lse = jnp.log2(_rowsum(jnp.exp2(y)))
        return y, lse

    @pl.when(s < nblk)
    def _pass1():
        x = xa_ref[:, :]
        y, lse = _y_lse(x, _rowsum(x * x))
        lt = _rowsum(jnp.where(cols == ta_ref[0, :, :], y, 0.0))
        acc_ref[0] += jnp.sum(lse - lt)
        gval = jnp.clip(lt * (1.0 / _C), 0.0, 1.0)
        g_ref[jax.lax.rem(s, 2)] = gval

        @pl.when(s == 0)
        def _pin():
            g_ref[2] = gval

    @pl.when(s > 0)
    def _pass2():
        gb = g_ref[jax.lax.rem(s - 1, 2)]
        gb1 = g_ref[jnp.where(s < nblk, jax.lax.rem(s, 2), 2)]
        g2 = jnp.concatenate([gb[_SHIFTS:, :], gb1[:_SHIFTS, :]], axis=0)
        lam = jnp.clip((gb + 1.0 - g2) * 0.5, 0.0, 1.0)
        hi = block - _SHIFTS
        lam_h = lam[:hi, :]
        virt_ref[:hi, :] = (lam_h * xb_ref[:hi, :] +
                            (1.0 - lam_h) * xb_ref[_SHIFTS:, :])
        lam_t = lam[hi:, :]
        virt_ref[hi:, :] = (lam_t * xb_ref[hi:, :] +
                            (1.0 - lam_t) * xa_ref[:_SHIFTS, :])
        virt = virt_ref[:, :]
        y, lse = _y_lse(virt, _rowsum(virt * virt))
        l1 = _rowsum(jnp.where(cols == tb_ref[0, :, :], y, 0.0))
        l2 = _rowsum(jnp.where(cols == t2b_ref[0, :, :], y, 0.0))
        acc_ref[1] += jnp.sum(lse - lam * l1 - (1.0 - lam) * l2)

    @pl.when(s == nblk)
    def _fin():
        out_ref[:, :] = jnp.full(
            (1, 1), _LN2 * (acc_ref[0] + acc_ref[1]) / (nblk * block),
            jnp.float32)


@functools.partial(jax.jit, static_argnames=("interpret",))
def kernel(X, T, proxies, interpret=False):
    n, e = X.shape
    ncls = proxies.shape[0]
    block = _BLOCK
    nblk = n // block

    T = T.astype(jnp.int32)
    t_col = T.reshape(nblk, block, 1)
    t2_col = jnp.roll(T, -_SHIFTS).reshape(nblk, block, 1)

    out = pl.pallas_call(
        functools.partial(_nca_body, nblk=nblk, block=block, ncls=ncls),
        grid=(nblk + 1,),
        in_specs=[
            pl.BlockSpec((block, e), lambda s: (jax.lax.rem(s, nblk), 0)),
            pl.BlockSpec((block, e), lambda s: (jnp.maximum(s - 1, 0), 0)),
            pl.BlockSpec((ncls, e), lambda s: (0, 0)),
            pl.BlockSpec((1, block, 1),
                         lambda s: (jax.lax.rem(s, nblk), 0, 0)),
            pl.BlockSpec((1, block, 1),
                         lambda s: (jnp.maximum(s - 1, 0), 0, 0)),
            pl.BlockSpec((1, block, 1),
                         lambda s: (jnp.maximum(s - 1, 0), 0, 0)),
        ],
        out_specs=pl.BlockSpec((1, 1), lambda s: (0, 0)),
        out_shape=jax.ShapeDtypeStruct((1, 1), jnp.float32),
        scratch_shapes=[
            pltpu.VMEM((ncls, e), jnp.bfloat16),
            pltpu.VMEM((e, 128), jnp.bfloat16),
            pltpu.VMEM((3, block, 1), jnp.float32),
            pltpu.VMEM((block, e), jnp.float32),
            pltpu.SMEM((2,), jnp.float32),
        ],
        interpret=interpret,
    )(X, X, proxies, t_col, t_col, t2_col)
    return out[0, 0]
